# Initial kernel scaffold; baseline (speedup 1.0000x reference)
import jax
import jax.numpy as jnp
from jax.experimental import pallas as pl


def kernel(r_ij, src, W_a, b_a, W_v, W_d):
    raise NotImplementedError("write your pallas kernel here")


# speedup vs baseline: 23.4796x; 23.4796x over previous
import functools

import jax
import jax.numpy as jnp
import numpy as np
from jax import lax
from jax.experimental import pallas as pl
from jax.experimental.pallas import tpu as pltpu
from jax.experimental.pallas import tpu_sc as plsc

_N = 50000
_E = 800000
_R0 = 6.0

_NW = 32
_KE = 1568
_EPT = 16 * _KE
_EP = _NW * _EPT
_NEB = _EPT // _KE
_NA = 50176
_KS = 6272
_NSB = 2 * _NA // _KS
_NPASS = 40
_EBT = 7168
_NB = 128


def _phi_body(r0_ref, r1_ref, r2_ref, *phi_refs):
    r0, r1, r2 = r0_ref[...], r1_ref[...], r2_ref[...]
    x_sq = (r0 * r0 + r1 * r1 + r2 * r2) / _R0
    arg = jnp.sqrt(x_sq + 1e-12)
    env = jnp.maximum(1.0 - x_sq, 0.0)
    sc17 = 17.0 / _R0
    y0, y1, y2 = r0 * sc17, r1 * sc17, r2 * sc17
    norm = jnp.sqrt(y0 * y0 + y1 * y1 + y2 * y2 + 1e-12)
    gate = (2.0 * jax.nn.sigmoid(norm) - 1.0) / norm
    rs0, rs1, rs2 = y0 * gate, y1 * gate, y2 * gate
    s10 = [jnp.ones_like(rs0), rs0, rs1, rs2, rs0 * rs0, rs0 * rs1,
           rs0 * rs2, rs1 * rs1, rs1 * rs2, rs2 * rs2]
    for c in range(8):
        rad_c = jnp.cos((np.pi * c) * arg) * env
        for k in range(10):
            phi_refs[c * 10 + k][...] = rad_c * s10[k]


def _phi_call(r0, r1, r2):
    return pl.pallas_call(
        _phi_body,
        grid=(_EP // _EBT,),
        in_specs=[pl.BlockSpec((_EBT,), lambda i: (i,))] * 3,
        out_specs=[pl.BlockSpec((_EBT,), lambda i: (i,))] * 80,
        out_shape=[jax.ShapeDtypeStruct((_EP,), jnp.float32)] * 80,
    )(r0, r1, r2)


@functools.cache
def _make_scatter_kernel():
    mesh = plsc.VectorSubcoreMesh(core_axis_name="c", subcore_axis_name="s")
    return functools.partial(
        pl.kernel,
        mesh=mesh,
        out_type=(jax.ShapeDtypeStruct((_NPASS, _NW, _NA), jnp.float32),
                  jax.ShapeDtypeStruct((_NPASS, _NW, _NA), jnp.float32)),
        scratch_types=[
            pltpu.VMEM((_KE,), jnp.int32),
            pltpu.VMEM((_KE,), jnp.int32),
            pltpu.VMEM((_KE,), jnp.float32),
            pltpu.VMEM((_KE,), jnp.float32),
            pltpu.VMEM((_KS,), jnp.int32),
            pltpu.VMEM((_KS,), jnp.float32),
            pltpu.VMEM((_KS,), jnp.float32),
            pltpu.VMEM_SHARED((16 * 2 * _NA,), jnp.float32),
        ],
    )(_scatter_body)


def _scatter_body(*args):
    phis = args[:80]
    src_a, src_b, sweep_idx, zeros_hbm, pa, pb = args[80:86]
    ia_v, ib_v, fa_v, fb_v, si_v, g_v, z_v, acc = args[86:]
    cid = lax.axis_index("c")
    sid = lax.axis_index("s")
    wid = sid * 2 + cid
    e0 = wid * _EPT
    sidx0 = sid * _EP
    sw0 = sid * (2 * _NA)

    pltpu.sync_copy(zeros_hbm, z_v)

    def zinit(i, c):
        pltpu.sync_copy(sweep_idx.at[pl.ds(sw0 + i * _KS, _KS)], si_v)
        pltpu.sync_copy(z_v, acc.at[si_v])
        return c

    lax.fori_loop(0, _NSB, zinit, 0)

    for p in range(_NPASS):
        phi_a, phi_b = phis[2 * p], phis[2 * p + 1]

        def ebody(i, c2, phi_a=phi_a, phi_b=phi_b):
            base = e0 + i * _KE
            pltpu.sync_copy(src_a.at[pl.ds(sidx0 + base, _KE)], ia_v)
            pltpu.sync_copy(src_b.at[pl.ds(sidx0 + base, _KE)], ib_v)
            pltpu.sync_copy(phi_a.at[pl.ds(base, _KE)], fa_v)
            pltpu.sync_copy(phi_b.at[pl.ds(base, _KE)], fb_v)
            pltpu.sync_copy(fa_v, acc.at[ia_v], add=True)
            pltpu.sync_copy(fb_v, acc.at[ib_v], add=True)
            return c2

        lax.fori_loop(0, _NEB, ebody, 0)

        def wbody(i, c2, p=p):
            pltpu.sync_copy(sweep_idx.at[pl.ds(sw0 + i * _KS, _KS)], si_v)
            pltpu.sync_copy(acc.at[si_v], g_v)
            pltpu.sync_copy(z_v, acc.at[si_v])

            @pl.when(i < _NSB // 2)
            def _():
                pltpu.sync_copy(g_v, pa.at[p, wid, pl.ds(i * _KS, _KS)])

            @pl.when(i >= _NSB // 2)
            def _():
                pltpu.sync_copy(
                    g_v, pb.at[p, wid, pl.ds(i * _KS - _NA, _KS)])
            return c2

        lax.fori_loop(0, _NSB, wbody, 0)


def _mix_body(pa_ref, pb_ref, wa_ref, ba_ref, wv_ref, wd_ref,
              oa_ref, ov_ref, od_ref):
    sa = jnp.sum(pa_ref[...], axis=1)
    sb = jnp.sum(pb_ref[...], axis=1)
    a_cat = jnp.concatenate([sa, sb], axis=0)
    dn = (((0,), (0,)), ((), ()))
    mm = lambda w_ref: lax.dot_general(
        a_cat, w_ref[...], dn, preferred_element_type=jnp.float32)
    oa_ref[...] = mm(wa_ref) + ba_ref[...]
    ov_ref[...] = mm(wv_ref)
    od_ref[...] = mm(wd_ref)


def _mix_call(pa, pb, wa, ba, wv, wd):
    full = lambda shape: pl.BlockSpec(shape, lambda i: (0, 0))
    grid = ((_N + _NB - 1) // _NB,)
    return pl.pallas_call(
        _mix_body,
        grid=grid,
        in_specs=[
            pl.BlockSpec((_NPASS, _NW, _NB), lambda i: (0, 0, i)),
            pl.BlockSpec((_NPASS, _NW, _NB), lambda i: (0, 0, i)),
            full((80, 128)), full((1, 128)), full((80, 384)),
            full((80, 1152)),
        ],
        out_specs=[
            pl.BlockSpec((_NB, 128), lambda i: (i, 0)),
            pl.BlockSpec((_NB, 384), lambda i: (i, 0)),
            pl.BlockSpec((_NB, 1152), lambda i: (i, 0)),
        ],
        out_shape=[
            jax.ShapeDtypeStruct((_N, 128), jnp.float32),
            jax.ShapeDtypeStruct((_N, 384), jnp.float32),
            jax.ShapeDtypeStruct((_N, 1152), jnp.float32),
        ],
    )(pa, pb, wa, ba, wv, wd)


def _expand_weights(W_a, b_a, W_v, W_d):
    wa = jnp.zeros((8, 10, 128), jnp.float32).at[:, 0, :].set(W_a)
    eye3 = jnp.eye(3, dtype=jnp.float32)
    wv = jnp.zeros((8, 10, 128, 3), jnp.float32).at[:, 1:4].set(
        jnp.einsum('co,ij->cioj', W_v, eye3))
    sym = np.zeros((6, 9), np.float32)
    for m, (i, j) in enumerate([(0, 0), (0, 1), (0, 2), (1, 1), (1, 2), (2, 2)]):
        sym[m, 3 * i + j] = 1.0
        sym[m, 3 * j + i] = 1.0
    wd = jnp.zeros((8, 10, 128, 9), jnp.float32).at[:, 4:10].set(
        jnp.einsum('co,mx->cmox', W_d, jnp.asarray(sym)))
    perm = np.concatenate([np.arange(0, 80, 2), np.arange(1, 80, 2)])
    return (wa.reshape(80, 128)[perm], b_a.reshape(1, 128),
            wv.reshape(80, 384)[perm], wd.reshape(80, 1152)[perm])


def kernel(r_ij, src, W_a, b_a, W_v, W_d):
    pad = jnp.zeros((_EP - _E,), jnp.float32)
    r0 = jnp.concatenate([r_ij[:, 0], pad])
    r1 = jnp.concatenate([r_ij[:, 1], pad])
    r2 = jnp.concatenate([r_ij[:, 2], pad])
    shift = (jnp.arange(16, dtype=jnp.int32) * (2 * _NA))[:, None]
    src0 = jnp.full((_EP,), _N, jnp.int32).at[:_E].set(src)
    src_a = (src0[None, :] + shift).reshape(-1)
    src_b = (src0[None, :] + _NA + shift).reshape(-1)
    sweep_idx = (jnp.arange(2 * _NA, dtype=jnp.int32)[None, :]
                 + shift).reshape(-1)
    zeros = jnp.zeros((_KS,), jnp.float32)

    phis = _phi_call(r0, r1, r2)
    pa, pb = _make_scatter_kernel()(*phis, src_a, src_b, sweep_idx, zeros)
    wa, ba, wv, wd = _expand_weights(W_a, b_a, W_v, W_d)
    oa, ov, od = _mix_call(pa, pb, wa, ba, wv, wd)
    return (oa, ov.reshape(_N, 128, 3), od.reshape(_N, 128, 3, 3))


# speedup vs baseline: 28.0425x; 1.1943x over previous
import functools

import jax
import jax.numpy as jnp
import numpy as np
from jax import lax
from jax.experimental import pallas as pl
from jax.experimental.pallas import tpu as pltpu
from jax.experimental.pallas import tpu_sc as plsc

_N = 50000
_E = 800000
_R0 = 6.0

_NW = 32
_KE = 1568
_EPT = 16 * _KE
_EP = _NW * _EPT
_NEB = _EPT // _KE
_NA = 50176
_KS = 6272
_NSB = 2 * _NA // _KS
_NPASS = 40
_EBT = 7168
_NB = 128


def _phi_body(r0_ref, r1_ref, r2_ref, *phi_refs):
    r0, r1, r2 = r0_ref[...], r1_ref[...], r2_ref[...]
    x_sq = (r0 * r0 + r1 * r1 + r2 * r2) / _R0
    arg = jnp.sqrt(x_sq + 1e-12)
    env = jnp.maximum(1.0 - x_sq, 0.0)
    sc17 = 17.0 / _R0
    y0, y1, y2 = r0 * sc17, r1 * sc17, r2 * sc17
    norm = jnp.sqrt(y0 * y0 + y1 * y1 + y2 * y2 + 1e-12)
    gate = (2.0 * jax.nn.sigmoid(norm) - 1.0) / norm
    rs0, rs1, rs2 = y0 * gate, y1 * gate, y2 * gate
    s10 = [jnp.ones_like(rs0), rs0, rs1, rs2, rs0 * rs0, rs0 * rs1,
           rs0 * rs2, rs1 * rs1, rs1 * rs2, rs2 * rs2]
    for c in range(8):
        rad_c = jnp.cos((np.pi * c) * arg) * env
        for k in range(10):
            phi_refs[c * 10 + k][...] = rad_c * s10[k]


def _phi_call(r0, r1, r2):
    return pl.pallas_call(
        _phi_body,
        grid=(_EP // _EBT,),
        in_specs=[pl.BlockSpec((_EBT,), lambda i: (i,))] * 3,
        out_specs=[pl.BlockSpec((_EBT,), lambda i: (i,))] * 80,
        out_shape=[jax.ShapeDtypeStruct((_EP,), jnp.float32)] * 80,
    )(r0, r1, r2)


@functools.cache
def _make_scatter_kernel():
    mesh = plsc.VectorSubcoreMesh(core_axis_name="c", subcore_axis_name="s")
    return functools.partial(
        pl.kernel,
        mesh=mesh,
        out_type=(jax.ShapeDtypeStruct((_NPASS, _NW, _NA), jnp.float32),
                  jax.ShapeDtypeStruct((_NPASS, _NW, _NA), jnp.float32)),
        scratch_types=[
            pltpu.VMEM((_KE,), jnp.int32),
            pltpu.VMEM((_KE,), jnp.int32),
            pltpu.VMEM((_KE,), jnp.float32),
            pltpu.VMEM((_KE,), jnp.float32),
            pltpu.VMEM((_KS,), jnp.int32),
            pltpu.VMEM((_KS,), jnp.float32),
            pltpu.VMEM((_KS,), jnp.float32),
            pltpu.VMEM_SHARED((16 * 2 * _NA,), jnp.float32),
            pltpu.SemaphoreType.DMA,
        ],
    )(_scatter_body)


def _scatter_body(*args):
    phis = args[:80]
    src_a, src_b, sweep_idx, zeros_hbm, pa, pb = args[80:86]
    ia_v, ib_v, fa_v, fb_v, si_v, g_v, z_v, acc, sem = args[86:]
    cid = lax.axis_index("c")
    sid = lax.axis_index("s")
    wid = sid * 2 + cid
    e0 = wid * _EPT
    sidx0 = sid * _EP
    sw0 = sid * (2 * _NA)

    pltpu.sync_copy(zeros_hbm, z_v)

    def zinit(i, c):
        pltpu.sync_copy(sweep_idx.at[pl.ds(sw0 + i * _KS, _KS)], si_v)
        pltpu.sync_copy(z_v, acc.at[si_v])
        return c

    lax.fori_loop(0, _NSB, zinit, 0)

    for p in range(_NPASS):
        phi_a, phi_b = phis[2 * p], phis[2 * p + 1]

        def ebody(i, c2, phi_a=phi_a, phi_b=phi_b):
            base = e0 + i * _KE
            c_ia = pltpu.async_copy(src_a.at[pl.ds(sidx0 + base, _KE)], ia_v, sem)
            c_ib = pltpu.async_copy(src_b.at[pl.ds(sidx0 + base, _KE)], ib_v, sem)
            c_fa = pltpu.async_copy(phi_a.at[pl.ds(base, _KE)], fa_v, sem)
            c_fb = pltpu.async_copy(phi_b.at[pl.ds(base, _KE)], fb_v, sem)
            c_ia.wait(); c_ib.wait(); c_fa.wait(); c_fb.wait()
            s_a = pltpu.async_copy(fa_v, acc.at[ia_v], sem, add=True)
            s_b = pltpu.async_copy(fb_v, acc.at[ib_v], sem, add=True)
            s_a.wait(); s_b.wait()
            return c2

        lax.fori_loop(0, _NEB, ebody, 0)

        def wbody(i, c2, p=p):
            pltpu.sync_copy(sweep_idx.at[pl.ds(sw0 + i * _KS, _KS)], si_v)
            pltpu.sync_copy(acc.at[si_v], g_v)
            rz = pltpu.async_copy(z_v, acc.at[si_v], sem)

            @pl.when(i < _NSB // 2)
            def _():
                pltpu.sync_copy(g_v, pa.at[p, wid, pl.ds(i * _KS, _KS)])

            @pl.when(i >= _NSB // 2)
            def _():
                pltpu.sync_copy(
                    g_v, pb.at[p, wid, pl.ds(i * _KS - _NA, _KS)])
            rz.wait()
            return c2

        lax.fori_loop(0, _NSB, wbody, 0)


def _mix_body(pa_ref, pb_ref, wa_ref, ba_ref, wv_ref, wd_ref,
              oa_ref, ov_ref, od_ref):
    sa = jnp.sum(pa_ref[...], axis=1)
    sb = jnp.sum(pb_ref[...], axis=1)
    a_cat = jnp.concatenate([sa, sb], axis=0)
    dn = (((0,), (0,)), ((), ()))
    mm = lambda w_ref: lax.dot_general(
        a_cat, w_ref[...], dn, preferred_element_type=jnp.float32)
    oa_ref[...] = mm(wa_ref) + ba_ref[...]
    ov_ref[...] = mm(wv_ref)
    od_ref[...] = mm(wd_ref)


def _mix_call(pa, pb, wa, ba, wv, wd):
    full = lambda shape: pl.BlockSpec(shape, lambda i: (0, 0))
    grid = ((_N + _NB - 1) // _NB,)
    return pl.pallas_call(
        _mix_body,
        grid=grid,
        in_specs=[
            pl.BlockSpec((_NPASS, _NW, _NB), lambda i: (0, 0, i)),
            pl.BlockSpec((_NPASS, _NW, _NB), lambda i: (0, 0, i)),
            full((80, 128)), full((1, 128)), full((80, 384)),
            full((80, 1152)),
        ],
        out_specs=[
            pl.BlockSpec((_NB, 128), lambda i: (i, 0)),
            pl.BlockSpec((_NB, 384), lambda i: (i, 0)),
            pl.BlockSpec((_NB, 1152), lambda i: (i, 0)),
        ],
        out_shape=[
            jax.ShapeDtypeStruct((_N, 128), jnp.float32),
            jax.ShapeDtypeStruct((_N, 384), jnp.float32),
            jax.ShapeDtypeStruct((_N, 1152), jnp.float32),
        ],
    )(pa, pb, wa, ba, wv, wd)


def _expand_weights(W_a, b_a, W_v, W_d):
    wa = jnp.zeros((8, 10, 128), jnp.float32).at[:, 0, :].set(W_a)
    eye3 = jnp.eye(3, dtype=jnp.float32)
    wv = jnp.zeros((8, 10, 128, 3), jnp.float32).at[:, 1:4].set(
        jnp.einsum('co,ij->cioj', W_v, eye3))
    sym = np.zeros((6, 9), np.float32)
    for m, (i, j) in enumerate([(0, 0), (0, 1), (0, 2), (1, 1), (1, 2), (2, 2)]):
        sym[m, 3 * i + j] = 1.0
        sym[m, 3 * j + i] = 1.0
    wd = jnp.zeros((8, 10, 128, 9), jnp.float32).at[:, 4:10].set(
        jnp.einsum('co,mx->cmox', W_d, jnp.asarray(sym)))
    perm = np.concatenate([np.arange(0, 80, 2), np.arange(1, 80, 2)])
    return (wa.reshape(80, 128)[perm], b_a.reshape(1, 128),
            wv.reshape(80, 384)[perm], wd.reshape(80, 1152)[perm])


def kernel(r_ij, src, W_a, b_a, W_v, W_d):
    pad = jnp.zeros((_EP - _E,), jnp.float32)
    r0 = jnp.concatenate([r_ij[:, 0], pad])
    r1 = jnp.concatenate([r_ij[:, 1], pad])
    r2 = jnp.concatenate([r_ij[:, 2], pad])
    shift = (jnp.arange(16, dtype=jnp.int32) * (2 * _NA))[:, None]
    src0 = jnp.full((_EP,), _N, jnp.int32).at[:_E].set(src)
    src_a = (src0[None, :] + shift).reshape(-1)
    src_b = (src0[None, :] + _NA + shift).reshape(-1)
    sweep_idx = (jnp.arange(2 * _NA, dtype=jnp.int32)[None, :]
                 + shift).reshape(-1)
    zeros = jnp.zeros((_KS,), jnp.float32)

    phis = _phi_call(r0, r1, r2)
    pa, pb = _make_scatter_kernel()(*phis, src_a, src_b, sweep_idx, zeros)
    wa, ba, wv, wd = _expand_weights(W_a, b_a, W_v, W_d)
    oa, ov, od = _mix_call(pa, pb, wa, ba, wv, wd)
    return (oa, ov.reshape(_N, 128, 3), od.reshape(_N, 128, 3, 3))
